# baseline (speedup 1.0000x reference)
import jax
import jax.numpy as jnp
from jax import lax
from jax.experimental import pallas as pl

_U = 16
_L = 128


def _cd_kernel(q_ref, dbe_ref, iv_ref, dist_ref, idx_ref):
    s = q_ref.shape[3]
    m = dbe_ref.shape[3]

    qx = q_ref[0, 0, 0]
    qy = q_ref[0, 0, 1]
    qz = q_ref[0, 0, 2]

    inf = jnp.full((s, _L), jnp.inf, jnp.float32)
    zero = jnp.zeros((s, _L), jnp.int32)

    def body(j, carry):
        ra_v, ra_i, rb_v, rb_i = carry
        base = j * _U
        pxb = dbe_ref[0, 0, 0, pl.ds(base, _U), :]
        pyb = dbe_ref[0, 0, 1, pl.ds(base, _U), :]
        pzb = dbe_ref[0, 0, 2, pl.ds(base, _U), :]
        ivb = iv_ref[pl.ds(base, _U), :]
        for k in range(_U):
            dx = qx - pxb[k:k + 1, :]
            dy = qy - pyb[k:k + 1, :]
            dz = qz - pzb[k:k + 1, :]
            d = dx * dx + dy * dy + dz * dz
            iv = ivb[k:k + 1, :]
            if k % 2 == 0:
                take = d < ra_v
                ra_v = jnp.where(take, d, ra_v)
                ra_i = jnp.where(take, iv, ra_i)
            else:
                take = d < rb_v
                rb_v = jnp.where(take, d, rb_v)
                rb_i = jnp.where(take, iv, rb_i)
        return ra_v, ra_i, rb_v, rb_i

    ra_v, ra_i, rb_v, rb_i = lax.fori_loop(
        0, m // _U, body, (inf, zero, inf, zero))
    bwin = (rb_v < ra_v) | ((rb_v == ra_v) & (rb_i < ra_i))
    dist_ref[0, 0] = jnp.where(bwin, rb_v, ra_v)
    idx_ref[0, 0] = jnp.where(bwin, rb_i, ra_i)


def _chamfer_both(q, dbe, iv):
    _, b, _, s, _ = q.shape
    m = dbe.shape[3]
    dist, idx = pl.pallas_call(
        _cd_kernel,
        grid=(2, b),
        in_specs=[
            pl.BlockSpec((1, 1, 3, s, _L), lambda p, bi: (p, bi, 0, 0, 0)),
            pl.BlockSpec((1, 1, 3, m, _L), lambda p, bi: (p, bi, 0, 0, 0)),
            pl.BlockSpec((m, _L), lambda p, bi: (0, 0)),
        ],
        out_specs=[
            pl.BlockSpec((1, 1, s, _L), lambda p, bi: (p, bi, 0, 0)),
            pl.BlockSpec((1, 1, s, _L), lambda p, bi: (p, bi, 0, 0)),
        ],
        out_shape=[
            jax.ShapeDtypeStruct((2, b, s, _L), jnp.float32),
            jax.ShapeDtypeStruct((2, b, s, _L), jnp.int32),
        ],
    )(q, dbe, iv)
    return dist, idx


def kernel(input1, input2):
    b, n, _ = input1.shape
    s = n // _L
    x1t = jnp.transpose(input1, (0, 2, 1))
    x2t = jnp.transpose(input2, (0, 2, 1))
    q = jnp.stack([x1t.reshape(b, 3, s, _L), x2t.reshape(b, 3, s, _L)])
    db = jnp.stack([x2t, x1t])
    dbe = jnp.broadcast_to(db[..., None], db.shape + (_L,))
    iv = jnp.broadcast_to(
        jnp.arange(n, dtype=jnp.int32)[:, None], (n, _L))
    dist, idx = _chamfer_both(q, dbe, iv)
    dist1 = dist[0].reshape(b, n)
    idx1 = idx[0].reshape(b, n)
    dist2 = dist[1].reshape(b, n)
    idx2 = idx[1].reshape(b, n)
    return (dist1, idx1, dist2, idx2)


# speedup vs baseline: 1.0006x; 1.0006x over previous
import jax
import jax.numpy as jnp
from jax import lax
from jax.experimental import pallas as pl

_U = 32
_L = 128


def _cd_kernel(q_ref, dbe_ref, iv_ref, dist_ref, idx_ref):
    s = q_ref.shape[3]
    m = dbe_ref.shape[3]

    qx = q_ref[0, 0, 0]
    qy = q_ref[0, 0, 1]
    qz = q_ref[0, 0, 2]

    inf = jnp.full((s, _L), jnp.inf, jnp.float32)
    zero = jnp.zeros((s, _L), jnp.int32)

    def body(j, carry):
        ra_v, ra_i, rb_v, rb_i = carry
        base = j * _U
        pxb = dbe_ref[0, 0, 0, pl.ds(base, _U), :]
        pyb = dbe_ref[0, 0, 1, pl.ds(base, _U), :]
        pzb = dbe_ref[0, 0, 2, pl.ds(base, _U), :]
        ivb = iv_ref[pl.ds(base, _U), :]
        for k in range(_U):
            dx = qx - pxb[k:k + 1, :]
            dy = qy - pyb[k:k + 1, :]
            dz = qz - pzb[k:k + 1, :]
            d = dx * dx + dy * dy + dz * dz
            iv = ivb[k:k + 1, :]
            if k % 2 == 0:
                take = d < ra_v
                ra_v = jnp.where(take, d, ra_v)
                ra_i = jnp.where(take, iv, ra_i)
            else:
                take = d < rb_v
                rb_v = jnp.where(take, d, rb_v)
                rb_i = jnp.where(take, iv, rb_i)
        return ra_v, ra_i, rb_v, rb_i

    ra_v, ra_i, rb_v, rb_i = lax.fori_loop(
        0, m // _U, body, (inf, zero, inf, zero))
    bwin = (rb_v < ra_v) | ((rb_v == ra_v) & (rb_i < ra_i))
    dist_ref[0, 0] = jnp.where(bwin, rb_v, ra_v)
    idx_ref[0, 0] = jnp.where(bwin, rb_i, ra_i)


def _chamfer_both(q, dbe, iv):
    _, b, _, s, _ = q.shape
    m = dbe.shape[3]
    dist, idx = pl.pallas_call(
        _cd_kernel,
        grid=(2, b),
        in_specs=[
            pl.BlockSpec((1, 1, 3, s, _L), lambda p, bi: (p, bi, 0, 0, 0)),
            pl.BlockSpec((1, 1, 3, m, _L), lambda p, bi: (p, bi, 0, 0, 0)),
            pl.BlockSpec((m, _L), lambda p, bi: (0, 0)),
        ],
        out_specs=[
            pl.BlockSpec((1, 1, s, _L), lambda p, bi: (p, bi, 0, 0)),
            pl.BlockSpec((1, 1, s, _L), lambda p, bi: (p, bi, 0, 0)),
        ],
        out_shape=[
            jax.ShapeDtypeStruct((2, b, s, _L), jnp.float32),
            jax.ShapeDtypeStruct((2, b, s, _L), jnp.int32),
        ],
    )(q, dbe, iv)
    return dist, idx


def kernel(input1, input2):
    b, n, _ = input1.shape
    s = n // _L
    x1t = jnp.transpose(input1, (0, 2, 1))
    x2t = jnp.transpose(input2, (0, 2, 1))
    q = jnp.stack([x1t.reshape(b, 3, s, _L), x2t.reshape(b, 3, s, _L)])
    db = jnp.stack([x2t, x1t])
    dbe = jnp.broadcast_to(db[..., None], db.shape + (_L,))
    iv = jnp.broadcast_to(
        jnp.arange(n, dtype=jnp.int32)[:, None], (n, _L))
    dist, idx = _chamfer_both(q, dbe, iv)
    dist1 = dist[0].reshape(b, n)
    idx1 = idx[0].reshape(b, n)
    dist2 = dist[1].reshape(b, n)
    idx2 = idx[1].reshape(b, n)
    return (dist1, idx1, dist2, idx2)


# speedup vs baseline: 1.0073x; 1.0067x over previous
import jax
import jax.numpy as jnp
from jax import lax
from jax.experimental import pallas as pl

_U = 16
_L = 128


def _cd_kernel(q_ref, dbp_ref, dist_ref, idx_ref):
    s = q_ref.shape[3]
    g = dbp_ref.shape[2]

    qx = q_ref[0, 0, 0]
    qy = q_ref[0, 0, 1]
    qz = q_ref[0, 0, 2]

    inf = jnp.full((s, _L), jnp.inf, jnp.float32)
    zero = jnp.zeros((s, _L), jnp.float32)

    def body(j, carry):
        rmin, ridx = carry
        blk = dbp_ref[0, 0, pl.ds(j, 1)]
        for k in range(_U):
            dx = qx - blk[0, k:k + 1, :]
            dy = qy - blk[0, _U + k:_U + k + 1, :]
            dz = qz - blk[0, 2 * _U + k:2 * _U + k + 1, :]
            d = dx * dx + dy * dy + dz * dz
            take = d < rmin
            rmin = jnp.where(take, d, rmin)
            ridx = jnp.where(take, blk[0, 3 * _U + k:3 * _U + k + 1, :], ridx)
        return rmin, ridx

    rmin, ridx = lax.fori_loop(0, g, body, (inf, zero))
    dist_ref[0, 0] = rmin
    idx_ref[0, 0] = ridx.astype(jnp.int32)


def _chamfer_both(q, dbp):
    _, b, _, s, _ = q.shape
    g = dbp.shape[2]
    dist, idx = pl.pallas_call(
        _cd_kernel,
        grid=(2, b),
        in_specs=[
            pl.BlockSpec((1, 1, 3, s, _L), lambda p, bi: (p, bi, 0, 0, 0)),
            pl.BlockSpec((1, 1, g, 4 * _U, _L),
                         lambda p, bi: (p, bi, 0, 0, 0)),
        ],
        out_specs=[
            pl.BlockSpec((1, 1, s, _L), lambda p, bi: (p, bi, 0, 0)),
            pl.BlockSpec((1, 1, s, _L), lambda p, bi: (p, bi, 0, 0)),
        ],
        out_shape=[
            jax.ShapeDtypeStruct((2, b, s, _L), jnp.float32),
            jax.ShapeDtypeStruct((2, b, s, _L), jnp.int32),
        ],
    )(q, dbp)
    return dist, idx


def kernel(input1, input2):
    b, n, _ = input1.shape
    s = n // _L
    g = n // _U
    x1t = jnp.transpose(input1, (0, 2, 1))
    x2t = jnp.transpose(input2, (0, 2, 1))
    q = jnp.stack([x1t.reshape(b, 3, s, _L), x2t.reshape(b, 3, s, _L)])
    db = jnp.stack([x2t, x1t])
    coords = db.reshape(2, b, 3, g, _U).transpose(0, 1, 3, 2, 4)
    ivf = jnp.broadcast_to(
        jnp.arange(n, dtype=jnp.float32).reshape(g, 1, _U), (g, 1, _U))
    ivf = jnp.broadcast_to(ivf[None, None], (2, b, g, 1, _U))
    dbp = jnp.concatenate([coords, ivf], axis=3)
    dbp = dbp.reshape(2, b, g, 4 * _U)
    dbp = jnp.broadcast_to(dbp[..., None], dbp.shape + (_L,))
    dist, idx = _chamfer_both(q, dbp)
    dist1 = dist[0].reshape(b, n)
    idx1 = idx[0].reshape(b, n)
    dist2 = dist[1].reshape(b, n)
    idx2 = idx[1].reshape(b, n)
    return (dist1, idx1, dist2, idx2)


# speedup vs baseline: 1.0192x; 1.0118x over previous
import jax
import jax.numpy as jnp
from jax import lax
from jax.experimental import pallas as pl

_U = 32
_L = 128


def _cd_kernel(q_ref, dbp_ref, dist_ref, idx_ref):
    s = q_ref.shape[3]
    g = dbp_ref.shape[2]

    qx = q_ref[0, 0, 0]
    qy = q_ref[0, 0, 1]
    qz = q_ref[0, 0, 2]

    inf = jnp.full((s, _L), jnp.inf, jnp.float32)
    zero = jnp.zeros((s, _L), jnp.float32)

    def body(j, carry):
        rmin, ridx = carry
        blk = dbp_ref[0, 0, pl.ds(j, 1)]
        for k in range(_U):
            dx = qx - blk[0, k:k + 1, :]
            dy = qy - blk[0, _U + k:_U + k + 1, :]
            dz = qz - blk[0, 2 * _U + k:2 * _U + k + 1, :]
            d = dx * dx + dy * dy + dz * dz
            take = d < rmin
            rmin = jnp.where(take, d, rmin)
            ridx = jnp.where(take, blk[0, 3 * _U + k:3 * _U + k + 1, :], ridx)
        return rmin, ridx

    rmin, ridx = lax.fori_loop(0, g, body, (inf, zero))
    dist_ref[0, 0] = rmin
    idx_ref[0, 0] = ridx.astype(jnp.int32)


def _chamfer_both(q, dbp):
    _, b, _, s, _ = q.shape
    g = dbp.shape[2]
    dist, idx = pl.pallas_call(
        _cd_kernel,
        grid=(2, b),
        in_specs=[
            pl.BlockSpec((1, 1, 3, s, _L), lambda p, bi: (p, bi, 0, 0, 0)),
            pl.BlockSpec((1, 1, g, 4 * _U, _L),
                         lambda p, bi: (p, bi, 0, 0, 0)),
        ],
        out_specs=[
            pl.BlockSpec((1, 1, s, _L), lambda p, bi: (p, bi, 0, 0)),
            pl.BlockSpec((1, 1, s, _L), lambda p, bi: (p, bi, 0, 0)),
        ],
        out_shape=[
            jax.ShapeDtypeStruct((2, b, s, _L), jnp.float32),
            jax.ShapeDtypeStruct((2, b, s, _L), jnp.int32),
        ],
    )(q, dbp)
    return dist, idx


def kernel(input1, input2):
    b, n, _ = input1.shape
    s = n // _L
    g = n // _U
    x1t = jnp.transpose(input1, (0, 2, 1))
    x2t = jnp.transpose(input2, (0, 2, 1))
    q = jnp.stack([x1t.reshape(b, 3, s, _L), x2t.reshape(b, 3, s, _L)])
    db = jnp.stack([x2t, x1t])
    coords = db.reshape(2, b, 3, g, _U).transpose(0, 1, 3, 2, 4)
    ivf = jnp.broadcast_to(
        jnp.arange(n, dtype=jnp.float32).reshape(g, 1, _U), (g, 1, _U))
    ivf = jnp.broadcast_to(ivf[None, None], (2, b, g, 1, _U))
    dbp = jnp.concatenate([coords, ivf], axis=3)
    dbp = dbp.reshape(2, b, g, 4 * _U)
    dbp = jnp.broadcast_to(dbp[..., None], dbp.shape + (_L,))
    dist, idx = _chamfer_both(q, dbp)
    dist1 = dist[0].reshape(b, n)
    idx1 = idx[0].reshape(b, n)
    dist2 = dist[1].reshape(b, n)
    idx2 = idx[1].reshape(b, n)
    return (dist1, idx1, dist2, idx2)


# speedup vs baseline: 1.0213x; 1.0021x over previous
import jax
import jax.numpy as jnp
from jax import lax
from jax.experimental import pallas as pl

_U = 64
_L = 128


def _cd_kernel(q_ref, dbp_ref, dist_ref, idx_ref):
    s = q_ref.shape[3]
    g = dbp_ref.shape[2]

    qx = q_ref[0, 0, 0]
    qy = q_ref[0, 0, 1]
    qz = q_ref[0, 0, 2]

    inf = jnp.full((s, _L), jnp.inf, jnp.float32)
    zero = jnp.zeros((s, _L), jnp.float32)

    def body(j, carry):
        rmin, ridx = carry
        blk = dbp_ref[0, 0, pl.ds(j, 1)]
        for k in range(_U):
            dx = qx - blk[0, k:k + 1, :]
            dy = qy - blk[0, _U + k:_U + k + 1, :]
            dz = qz - blk[0, 2 * _U + k:2 * _U + k + 1, :]
            d = dx * dx + dy * dy + dz * dz
            take = d < rmin
            rmin = jnp.where(take, d, rmin)
            ridx = jnp.where(take, blk[0, 3 * _U + k:3 * _U + k + 1, :], ridx)
        return rmin, ridx

    rmin, ridx = lax.fori_loop(0, g, body, (inf, zero))
    dist_ref[0, 0] = rmin
    idx_ref[0, 0] = ridx.astype(jnp.int32)


def _chamfer_both(q, dbp):
    _, b, _, s, _ = q.shape
    g = dbp.shape[2]
    dist, idx = pl.pallas_call(
        _cd_kernel,
        grid=(2, b),
        in_specs=[
            pl.BlockSpec((1, 1, 3, s, _L), lambda p, bi: (p, bi, 0, 0, 0)),
            pl.BlockSpec((1, 1, g, 4 * _U, _L),
                         lambda p, bi: (p, bi, 0, 0, 0)),
        ],
        out_specs=[
            pl.BlockSpec((1, 1, s, _L), lambda p, bi: (p, bi, 0, 0)),
            pl.BlockSpec((1, 1, s, _L), lambda p, bi: (p, bi, 0, 0)),
        ],
        out_shape=[
            jax.ShapeDtypeStruct((2, b, s, _L), jnp.float32),
            jax.ShapeDtypeStruct((2, b, s, _L), jnp.int32),
        ],
    )(q, dbp)
    return dist, idx


def kernel(input1, input2):
    b, n, _ = input1.shape
    s = n // _L
    g = n // _U
    x1t = jnp.transpose(input1, (0, 2, 1))
    x2t = jnp.transpose(input2, (0, 2, 1))
    q = jnp.stack([x1t.reshape(b, 3, s, _L), x2t.reshape(b, 3, s, _L)])
    db = jnp.stack([x2t, x1t])
    coords = db.reshape(2, b, 3, g, _U).transpose(0, 1, 3, 2, 4)
    ivf = jnp.broadcast_to(
        jnp.arange(n, dtype=jnp.float32).reshape(g, 1, _U), (g, 1, _U))
    ivf = jnp.broadcast_to(ivf[None, None], (2, b, g, 1, _U))
    dbp = jnp.concatenate([coords, ivf], axis=3)
    dbp = dbp.reshape(2, b, g, 4 * _U)
    dbp = jnp.broadcast_to(dbp[..., None], dbp.shape + (_L,))
    dist, idx = _chamfer_both(q, dbp)
    dist1 = dist[0].reshape(b, n)
    idx1 = idx[0].reshape(b, n)
    dist2 = dist[1].reshape(b, n)
    idx2 = idx[1].reshape(b, n)
    return (dist1, idx1, dist2, idx2)
